# baseline (speedup 1.0000x reference)
import functools

import jax
import jax.numpy as jnp
from jax import lax
from jax.experimental import pallas as pl
from jax.experimental.pallas import tpu as pltpu
from jax.experimental.pallas import tpu_sc as plsc

N = 10000
D = 256
HALF = 128
E = 160000
NS = 16
NC = 2
CH = 128
NCHUNK = 82
EPT = NCHUNK * CH
E_PAD = NS * EPT
ACC_ROWS = 10240
TRASH = N
RPT = ACC_ROWS // NS
WCH = 128
ROW_BLK = 512
NBUF = 2
NBK = ACC_ROWS // ROW_BLK
NGRID = (N + ROW_BLK - 1) // ROW_BLK


def _mm_body(x_ref, w_ref, o_ref):
    o_ref[...] = jnp.dot(x_ref[...], w_ref[...],
                         preferred_element_type=jnp.float32)


def _mm_split(x, w):
    return pl.pallas_call(
        _mm_body,
        grid=(NC, NGRID),
        in_specs=[
            pl.BlockSpec((ROW_BLK, D), lambda h, i: (i, 0)),
            pl.BlockSpec((D, HALF), lambda h, i: (0, h)),
        ],
        out_specs=pl.BlockSpec((ROW_BLK, HALF), lambda h, i: (h * NBK + i, 0)),
        out_shape=jax.ShapeDtypeStruct((NC * ACC_ROWS, HALF), jnp.float32),
    )(x, w)


def _act_body(s0_ref, s1_ref, deg_ref, x_ref, wr_ref, b_ref, o_ref, *, relu):
    r = 1.0 / jnp.maximum(deg_ref[...], 1.0)
    s = jnp.concatenate([s0_ref[...] * r, s1_ref[...] * r], axis=1)
    y = s + b_ref[...] + jnp.dot(x_ref[...], wr_ref[...],
                                 preferred_element_type=jnp.float32)
    if relu:
        y = jnp.maximum(y, 0.0)
    o_ref[...] = y


def _act(seg, degs, x, wr, b2d, t_seg, t_deg, relu):
    return pl.pallas_call(
        functools.partial(_act_body, relu=relu),
        grid=(NGRID,),
        in_specs=[
            pl.BlockSpec((ROW_BLK, HALF), lambda i: (i + t_seg * NBK, 0)),
            pl.BlockSpec((ROW_BLK, HALF), lambda i: (i + (t_seg + 1) * NBK, 0)),
            pl.BlockSpec((ROW_BLK, HALF), lambda i: (i + t_deg * NBK, 0)),
            pl.BlockSpec((ROW_BLK, D), lambda i: (i, 0)),
            pl.BlockSpec((D, D), lambda i: (0, 0)),
            pl.BlockSpec((1, D), lambda i: (0, 0)),
        ],
        out_specs=pl.BlockSpec((ROW_BLK, D), lambda i: (i, 0)),
        out_shape=jax.ShapeDtypeStruct((N, D), jnp.float32),
    )(seg, seg, degs, x, wr, b2d)


NBUF = 2
GDEPTH = NBUF - 1
MASK15 = 0x7FFF


def _zero_rows(rows):
    def zb(i, carry):
        rows[i // 8, pl.ds((i % 8) * 16, 16)] = jnp.zeros((16,), jnp.float32)
        return carry
    lax.fori_loop(0, (WCH * HALF) // 16, zb, 0)


def _zero_acc(rows, acc, s):
    def zc(k, carry):
        pltpu.sync_copy(rows, acc.at[pl.ds(s * RPT + k * WCH, WCH)])
        return carry
    lax.fori_loop(0, RPT // WCH, zc, 0)


def _writeout(rows, acc, out, base, s):
    def wo(k, carry):
        r0 = s * RPT + k * WCH
        pltpu.sync_copy(acc.at[pl.ds(r0, WCH)], rows)
        pltpu.sync_copy(rows, out.at[pl.ds(base + r0, WCH)])
        return carry
    lax.fori_loop(0, RPT // WCH, wo, 0)


def _unpack_idx(pk, j, sb, db):
    def up(q, carry):
        v = pk[j, pl.ds(q * 16, 16)]
        sb[0, pl.ds(q * 16, 16)] = v & MASK15
        db[0, pl.ds(q * 16, 16)] = lax.shift_right_logical(v, 15)
        return carry
    lax.fori_loop(0, CH // 16, up, 0)


def _seg_pass(ytab, pks, out, base, w, s, pk, sbs, dbs, rows, gs, acc):
    pltpu.sync_copy(pks.at[w], pk)
    _zero_rows(rows[0])
    _zero_acc(rows[0], acc, s)
    plsc.subcore_barrier()

    for t in range(GDEPTH):
        _unpack_idx(pk, t, sbs[t], dbs[t])
        pltpu.async_copy(ytab.at[sbs[t].at[0]], rows[t], gs[t])

    def grp(k, carry):
        for t in range(NBUF):
            j = NBUF * k + t
            jn = j + GDEPTH
            tn = (t + GDEPTH) % NBUF

            @pl.when(jn < NCHUNK)
            def _():
                _unpack_idx(pk, jn, sbs[tn], dbs[tn])
                pltpu.async_copy(ytab.at[sbs[tn].at[0]], rows[tn], gs[tn])

            pltpu.make_async_copy(ytab.at[sbs[t].at[0]], rows[t], gs[t]).wait()
            pltpu.sync_copy(rows[t], acc.at[dbs[t].at[0]], add=True)
        return carry
    lax.fori_loop(0, NCHUNK // NBUF, grp, 0)
    plsc.subcore_barrier()
    _writeout(rows[0], acc, out, base, s)


def _deg_pass(pks, consts, w, s, pk, sb, db, r0, acc):
    pltpu.sync_copy(pks.at[w], pk)
    pltpu.sync_copy(consts.at[0], r0)
    _zero_acc(r0, acc, s)
    pltpu.sync_copy(consts.at[1], r0)
    plsc.subcore_barrier()

    def dstep(j, carry):
        _unpack_idx(pk, j, sb, db)
        pltpu.sync_copy(r0, acc.at[sb.at[0]], add=True)
        return carry
    lax.fori_loop(0, NCHUNK, dstep, 0)
    plsc.subcore_barrier()


def _sc1_body(ytabA, ytabB, pkA, pkB, pkD, consts, outseg, outdeg,
              r0, r1, sb0, sb1, db0, db1, pk, acc, g0, g1):
    c = lax.axis_index("c")
    s = lax.axis_index("s")
    w = c * NS + s
    rows = (r0, r1)
    sbs = (sb0, sb1)
    dbs = (db0, db1)
    gs = (g0, g1)
    _seg_pass(ytabA, pkA, outseg, c * ACC_ROWS, w, s, pk, sbs, dbs, rows, gs, acc)
    _seg_pass(ytabB, pkB, outseg, (2 + c) * ACC_ROWS, w, s, pk, sbs, dbs, rows, gs, acc)
    _deg_pass(pkD, consts, w, s, pk, sb0, db0, r0, acc)
    _writeout(r0, acc, outdeg, c * ACC_ROWS, s)


def _sc2_body(ytabA, ytabB, pkA, pkB, outseg,
              r0, r1, sb0, sb1, db0, db1, pk, acc, g0, g1):
    c = lax.axis_index("c")
    s = lax.axis_index("s")
    w = c * NS + s
    rows = (r0, r1)
    sbs = (sb0, sb1)
    dbs = (db0, db1)
    gs = (g0, g1)
    _seg_pass(ytabA, pkA, outseg, c * ACC_ROWS, w, s, pk, sbs, dbs, rows, gs, acc)
    _seg_pass(ytabB, pkB, outseg, (2 + c) * ACC_ROWS, w, s, pk, sbs, dbs, rows, gs, acc)


_SC_SCRATCH = (
    [pltpu.VMEM((WCH, HALF), jnp.float32)] * NBUF
    + [pltpu.VMEM((1, CH), jnp.int32)] * (2 * NBUF)
    + [pltpu.VMEM((NCHUNK, CH), jnp.int32)]
    + [pltpu.VMEM_SHARED((ACC_ROWS, HALF), jnp.float32)]
    + [pltpu.SemaphoreType.DMA] * NBUF
)

_sc1_call = pl.kernel(
    _sc1_body,
    out_type=(jax.ShapeDtypeStruct((4 * ACC_ROWS, HALF), jnp.float32),
              jax.ShapeDtypeStruct((2 * ACC_ROWS, HALF), jnp.float32)),
    mesh=plsc.VectorSubcoreMesh(core_axis_name="c", subcore_axis_name="s"),
    scratch_types=_SC_SCRATCH,
)

_sc2_call = pl.kernel(
    _sc2_body,
    out_type=jax.ShapeDtypeStruct((4 * ACC_ROWS, HALF), jnp.float32),
    mesh=plsc.VectorSubcoreMesh(core_axis_name="c", subcore_axis_name="s"),
    scratch_types=_SC_SCRATCH,
)


def _prep_idx(idx, fill):
    p = jnp.full((E_PAD - E,), fill, jnp.int32)
    return jnp.concatenate([idx.astype(jnp.int32), p]).reshape(NS, NCHUNK, CH)


def _pack(src, dst):
    return jnp.concatenate([src | (dst << 15),
                            (src + ACC_ROWS) | (dst << 15)], axis=0)


def kernel(x_user, x_item, edge_index_ui, edge_index_iu,
           W1_ui_l, b1_ui, W1_ui_r, W1_iu_l, b1_iu, W1_iu_r,
           W2_ui_l, b2_ui, W2_ui_r, W2_iu_l, b2_iu, W2_iu_r):
    src_ui = _prep_idx(edge_index_ui[0], 0)
    dst_ui = _prep_idx(edge_index_ui[1], TRASH)
    src_iu = _prep_idx(edge_index_iu[0], 0)
    dst_iu = _prep_idx(edge_index_iu[1], TRASH)
    pk_ui = _pack(src_ui, dst_ui)
    pk_iu = _pack(src_iu, dst_iu)
    pk_deg = jnp.concatenate([dst_ui, dst_iu], axis=0)
    consts = jnp.stack([jnp.zeros((CH, HALF), jnp.float32),
                        jnp.ones((CH, HALF), jnp.float32)])

    y1u = _mm_split(x_user, W1_ui_l)
    y1i = _mm_split(x_item, W1_iu_l)
    seg1, degs = _sc1_call(y1u, y1i, pk_ui, pk_iu, pk_deg, consts)
    h_item = _act(seg1, degs, x_item, W1_ui_r, b1_ui.reshape(1, D),
                  t_seg=0, t_deg=0, relu=True)
    h_user = _act(seg1, degs, x_user, W1_iu_r, b1_iu.reshape(1, D),
                  t_seg=2, t_deg=1, relu=True)

    y2u = _mm_split(h_user, W2_ui_l)
    y2i = _mm_split(h_item, W2_iu_l)
    seg2 = _sc2_call(y2u, y2i, pk_ui, pk_iu)
    o_item = _act(seg2, degs, h_item, W2_ui_r, b2_ui.reshape(1, D),
                  t_seg=0, t_deg=0, relu=False)
    o_user = _act(seg2, degs, h_user, W2_iu_r, b2_iu.reshape(1, D),
                  t_seg=2, t_deg=1, relu=False)
    return (o_user, o_item)


# speedup vs baseline: 1.3118x; 1.3118x over previous
import functools

import jax
import jax.numpy as jnp
from jax import lax
from jax.experimental import pallas as pl
from jax.experimental.pallas import tpu as pltpu
from jax.experimental.pallas import tpu_sc as plsc

N = 10000
D = 256
HALF = 128
E = 160000
NS = 16
NC = 2
CH = 128
NCHUNK = 80
EPT = NCHUNK * CH
E_PAD = NS * EPT
ACC_ROWS = 10240
TRASH = N
RPT = ACC_ROWS // NS
WCH = 128
ROW_BLK = 512
NBUF = 2
NBK = ACC_ROWS // ROW_BLK
NGRID = (N + ROW_BLK - 1) // ROW_BLK


def _mm_body(x_ref, w_ref, o_ref):
    o_ref[...] = jnp.dot(x_ref[...], w_ref[...],
                         preferred_element_type=jnp.float32)


def _mm_split(x, w):
    return pl.pallas_call(
        _mm_body,
        grid=(NC, NGRID),
        in_specs=[
            pl.BlockSpec((ROW_BLK, D), lambda h, i: (i, 0)),
            pl.BlockSpec((D, HALF), lambda h, i: (0, h)),
        ],
        out_specs=pl.BlockSpec((ROW_BLK, HALF), lambda h, i: (h * NBK + i, 0)),
        out_shape=jax.ShapeDtypeStruct((NC * ACC_ROWS, HALF), jnp.float32),
    )(x, w)


def _act_body(s0_ref, s1_ref, deg_ref, x_ref, wr_ref, b_ref, o_ref, *, relu):
    r = 1.0 / jnp.maximum(deg_ref[...], 1.0)
    s = jnp.concatenate([s0_ref[...] * r, s1_ref[...] * r], axis=1)
    y = s + b_ref[...] + jnp.dot(x_ref[...], wr_ref[...],
                                 preferred_element_type=jnp.float32)
    if relu:
        y = jnp.maximum(y, 0.0)
    o_ref[...] = y


def _act(seg, degs, x, wr, b2d, t_seg, t_deg, relu):
    return pl.pallas_call(
        functools.partial(_act_body, relu=relu),
        grid=(NGRID,),
        in_specs=[
            pl.BlockSpec((ROW_BLK, HALF), lambda i: (i + t_seg * NBK, 0)),
            pl.BlockSpec((ROW_BLK, HALF), lambda i: (i + (t_seg + 1) * NBK, 0)),
            pl.BlockSpec((ROW_BLK, HALF), lambda i: (i + t_deg * NBK, 0)),
            pl.BlockSpec((ROW_BLK, D), lambda i: (i, 0)),
            pl.BlockSpec((D, D), lambda i: (0, 0)),
            pl.BlockSpec((1, D), lambda i: (0, 0)),
        ],
        out_specs=pl.BlockSpec((ROW_BLK, D), lambda i: (i, 0)),
        out_shape=jax.ShapeDtypeStruct((N, D), jnp.float32),
    )(seg, seg, degs, x, wr, b2d)


NBUF = 2
GDEPTH = NBUF - 1
MASK15 = 0x7FFF


def _zero_rows(rows):
    def zb(i, carry):
        rows[i // 8, pl.ds((i % 8) * 16, 16)] = jnp.zeros((16,), jnp.float32)
        return carry
    lax.fori_loop(0, (WCH * HALF) // 16, zb, 0)


def _zero_acc(rows, acc, s):
    def zc(k, carry):
        pltpu.sync_copy(rows, acc.at[pl.ds(s * RPT + k * WCH, WCH)])
        return carry
    lax.fori_loop(0, RPT // WCH, zc, 0)


def _writeout(rows, acc, out, base, s):
    def wo(k, carry):
        r0 = s * RPT + k * WCH
        pltpu.sync_copy(acc.at[pl.ds(r0, WCH)], rows)
        pltpu.sync_copy(rows, out.at[pl.ds(base + r0, WCH)])
        return carry
    lax.fori_loop(0, RPT // WCH, wo, 0)


def _unpack_idx(pk, j, sb, db):
    for q in range(CH // 16):
        v = pk[j, pl.ds(q * 16, 16)]
        sb[0, pl.ds(q * 16, 16)] = v & MASK15
        db[0, pl.ds(q * 16, 16)] = lax.shift_right_logical(v, 15)


def _seg_pass(ytab, pks, out, base, w, s, pk, sbs, dbs, rows, gs, acc):
    pltpu.sync_copy(pks.at[w], pk)
    _zero_rows(rows[0])
    _zero_acc(rows[0], acc, s)
    plsc.subcore_barrier()

    def grp(k, carry):
        _unpack_idx(pk, 2 * k, sbs[0], dbs[0])
        _unpack_idx(pk, 2 * k + 1, sbs[1], dbs[1])
        d0 = pltpu.async_copy(ytab.at[sbs[0].at[0]], rows[0], gs[0])
        d1 = pltpu.async_copy(ytab.at[sbs[1].at[0]], rows[1], gs[1])
        d0.wait()
        pltpu.sync_copy(rows[0], acc.at[dbs[0].at[0]], add=True)
        d1.wait()
        pltpu.sync_copy(rows[1], acc.at[dbs[1].at[0]], add=True)
        return carry
    lax.fori_loop(0, NCHUNK // 2, grp, 0)
    plsc.subcore_barrier()
    _writeout(rows[0], acc, out, base, s)


def _deg_pass(pks, consts, w, s, pk, sb, db, r0, acc):
    pltpu.sync_copy(pks.at[w], pk)
    pltpu.sync_copy(consts.at[0], r0)
    _zero_acc(r0, acc, s)
    pltpu.sync_copy(consts.at[1], r0)
    plsc.subcore_barrier()

    def dstep(j, carry):
        _unpack_idx(pk, j, sb, db)
        pltpu.sync_copy(r0, acc.at[sb.at[0]], add=True)
        return carry
    lax.fori_loop(0, NCHUNK, dstep, 0)
    plsc.subcore_barrier()


def _sc1_body(ytabA, ytabB, pkA, pkB, pkD, consts, outseg, outdeg,
              r0, r1, sb0, sb1, db0, db1, pk, acc, g0, g1):
    c = lax.axis_index("c")
    s = lax.axis_index("s")
    w = c * NS + s
    rows = (r0, r1)
    sbs = (sb0, sb1)
    dbs = (db0, db1)
    gs = (g0, g1)
    _seg_pass(ytabA, pkA, outseg, c * ACC_ROWS, w, s, pk, sbs, dbs, rows, gs, acc)
    _seg_pass(ytabB, pkB, outseg, (2 + c) * ACC_ROWS, w, s, pk, sbs, dbs, rows, gs, acc)
    _deg_pass(pkD, consts, w, s, pk, sb0, db0, r0, acc)
    _writeout(r0, acc, outdeg, c * ACC_ROWS, s)


def _sc2_body(ytabA, ytabB, pkA, pkB, outseg,
              r0, r1, sb0, sb1, db0, db1, pk, acc, g0, g1):
    c = lax.axis_index("c")
    s = lax.axis_index("s")
    w = c * NS + s
    rows = (r0, r1)
    sbs = (sb0, sb1)
    dbs = (db0, db1)
    gs = (g0, g1)
    _seg_pass(ytabA, pkA, outseg, c * ACC_ROWS, w, s, pk, sbs, dbs, rows, gs, acc)
    _seg_pass(ytabB, pkB, outseg, (2 + c) * ACC_ROWS, w, s, pk, sbs, dbs, rows, gs, acc)


_SC_SCRATCH = (
    [pltpu.VMEM((WCH, HALF), jnp.float32)] * NBUF
    + [pltpu.VMEM((1, CH), jnp.int32)] * (2 * NBUF)
    + [pltpu.VMEM((NCHUNK, CH), jnp.int32)]
    + [pltpu.VMEM_SHARED((ACC_ROWS, HALF), jnp.float32)]
    + [pltpu.SemaphoreType.DMA] * NBUF
)

_sc1_call = pl.kernel(
    _sc1_body,
    out_type=(jax.ShapeDtypeStruct((4 * ACC_ROWS, HALF), jnp.float32),
              jax.ShapeDtypeStruct((2 * ACC_ROWS, HALF), jnp.float32)),
    mesh=plsc.VectorSubcoreMesh(core_axis_name="c", subcore_axis_name="s"),
    scratch_types=_SC_SCRATCH,
)

_sc2_call = pl.kernel(
    _sc2_body,
    out_type=jax.ShapeDtypeStruct((4 * ACC_ROWS, HALF), jnp.float32),
    mesh=plsc.VectorSubcoreMesh(core_axis_name="c", subcore_axis_name="s"),
    scratch_types=_SC_SCRATCH,
)


def _prep_idx(idx, fill):
    p = jnp.full((E_PAD - E,), fill, jnp.int32)
    return jnp.concatenate([idx.astype(jnp.int32), p]).reshape(NS, NCHUNK, CH)


def _pack(src, dst):
    return jnp.concatenate([src | (dst << 15),
                            (src + ACC_ROWS) | (dst << 15)], axis=0)


def kernel(x_user, x_item, edge_index_ui, edge_index_iu,
           W1_ui_l, b1_ui, W1_ui_r, W1_iu_l, b1_iu, W1_iu_r,
           W2_ui_l, b2_ui, W2_ui_r, W2_iu_l, b2_iu, W2_iu_r):
    src_ui = _prep_idx(edge_index_ui[0], 0)
    dst_ui = _prep_idx(edge_index_ui[1], TRASH)
    src_iu = _prep_idx(edge_index_iu[0], 0)
    dst_iu = _prep_idx(edge_index_iu[1], TRASH)
    pk_ui = _pack(src_ui, dst_ui)
    pk_iu = _pack(src_iu, dst_iu)
    pk_deg = jnp.concatenate([dst_ui, dst_iu], axis=0)
    consts = jnp.stack([jnp.zeros((CH, HALF), jnp.float32),
                        jnp.ones((CH, HALF), jnp.float32)])

    y1u = _mm_split(x_user, W1_ui_l)
    y1i = _mm_split(x_item, W1_iu_l)
    seg1, degs = _sc1_call(y1u, y1i, pk_ui, pk_iu, pk_deg, consts)
    h_item = _act(seg1, degs, x_item, W1_ui_r, b1_ui.reshape(1, D),
                  t_seg=0, t_deg=0, relu=True)
    h_user = _act(seg1, degs, x_user, W1_iu_r, b1_iu.reshape(1, D),
                  t_seg=2, t_deg=1, relu=True)

    y2u = _mm_split(h_user, W2_ui_l)
    y2i = _mm_split(h_item, W2_iu_l)
    seg2 = _sc2_call(y2u, y2i, pk_ui, pk_iu)
    o_item = _act(seg2, degs, h_item, W2_ui_r, b2_ui.reshape(1, D),
                  t_seg=0, t_deg=0, relu=False)
    o_user = _act(seg2, degs, h_user, W2_iu_r, b2_iu.reshape(1, D),
                  t_seg=2, t_deg=1, relu=False)
    return (o_user, o_item)
